# Initial kernel scaffold; baseline (speedup 1.0000x reference)
import jax
import jax.numpy as jnp
from jax.experimental import pallas as pl


def kernel(x, edge_index, W1, b1, g1, be1, W2, b2, g2, be2, W3, b3, g3, be3, W4, b4, g4, be4, Wo, bo):
    raise NotImplementedError("write your pallas kernel here")


# speedup vs baseline: 26.3832x; 26.3832x over previous
import functools

import jax
import jax.numpy as jnp
from jax import lax
from jax.experimental import pallas as pl
from jax.experimental.pallas import tpu as pltpu
from jax.experimental.pallas import tpu_sc as plsc

F32 = jnp.float32
EPS = 1e-5

NC = 2
NS = 16
NW = NC * NS
CH = 128
GRP = 8
D = 16

def _sc_agg_call(nrows, ngrp):
    zr = nrows // NS
    mesh = plsc.VectorSubcoreMesh(core_axis_name="c", subcore_axis_name="s")

    @functools.partial(
        pl.kernel,
        out_type=jax.ShapeDtypeStruct((NC, nrows, D), F32),
        mesh=mesh,
        scratch_types=[
            pltpu.VMEM_SHARED((nrows, D), F32),
            pltpu.VMEM((GRP, CH), jnp.int32),
            pltpu.VMEM((GRP, CH), jnp.int32),
            pltpu.VMEM((GRP, CH, D), F32),
            pltpu.VMEM((zr, D), F32),
            pltpu.SemaphoreType.DMA,
        ],
        compiler_params=pltpu.CompilerParams(use_tc_tiling_on_sc=False),
    )
    def agg(table, src3, dst3, out, acc, sidx, didx, rows, zb, sem):
        c = lax.axis_index("c")
        s = lax.axis_index("s")
        w = c * NS + s

        def zfill(i, _):
            zb[i] = jnp.zeros((D,), F32)
            return 0

        lax.fori_loop(0, zr, zfill, 0)
        row0 = pl.multiple_of(s * zr, 8)
        pltpu.sync_copy(zb, acc.at[pl.ds(row0, zr)])
        plsc.subcore_barrier()

        def group(g, _):
            pltpu.sync_copy(src3.at[w, pl.ds(g * GRP, GRP)], sidx)
            pltpu.sync_copy(dst3.at[w, pl.ds(g * GRP, GRP)], didx)
            descs = [
                pltpu.async_copy(table.at[sidx.at[j]], rows.at[j], sem)
                for j in range(GRP)
            ]
            for j in range(GRP):
                descs[j].wait()
                pltpu.sync_copy(rows.at[j], acc.at[didx.at[j]], add=True)
            return 0

        lax.fori_loop(0, ngrp, group, 0)
        plsc.subcore_barrier()
        pltpu.sync_copy(acc.at[pl.ds(row0, zr)], out.at[c, pl.ds(row0, zr)])

    return agg


def _pre_body(x_ref, w1_ref, degp_ref, m_ref, dinv_ref):
    n = m_ref.shape[0]
    deg = degp_ref[0, :n, 0:1] + degp_ref[1, :n, 0:1] + 1.0
    dinv = lax.rsqrt(deg)
    h = jnp.dot(x_ref[...], w1_ref[...], preferred_element_type=F32)
    dinv_ref[...] = dinv
    m_ref[...] = h * dinv


def _bn_relu(aggp_ref, m_ref, dinv_ref, b_ref, g_ref, be_ref):
    n = m_ref.shape[0]
    dinv = dinv_ref[...]
    t = (aggp_ref[0, :n] + aggp_ref[1, :n] + m_ref[...]) * dinv + b_ref[...]
    mu = jnp.mean(t, axis=0, keepdims=True)
    var = jnp.mean((t - mu) ** 2, axis=0, keepdims=True)
    return jnp.maximum((t - mu) * lax.rsqrt(var + EPS) * g_ref[...] + be_ref[...], 0.0)


def _mid_body(aggp_ref, m_ref, dinv_ref, b_ref, g_ref, be_ref, w_ref, out_ref):
    y = _bn_relu(aggp_ref, m_ref, dinv_ref, b_ref, g_ref, be_ref)
    out_ref[...] = jnp.dot(y, w_ref[...], preferred_element_type=F32) * dinv_ref[...]


def _fin_body(aggp_ref, m_ref, dinv_ref, b_ref, g_ref, be_ref, wo_ref, bo_ref, out_ref):
    y = _bn_relu(aggp_ref, m_ref, dinv_ref, b_ref, g_ref, be_ref)
    pooled = jnp.max(y, axis=0, keepdims=True)
    out_ref[...] = jnp.sum(pooled * wo_ref[...], axis=1, keepdims=True) + bo_ref[...]


def _padw(w, rows=D, cols=D):
    return jnp.pad(w, ((0, rows - w.shape[0]), (0, cols - w.shape[1])))


def _padv(v):
    return jnp.pad(v, (0, D - v.shape[0])).reshape(1, D)


def kernel(x, edge_index, W1, b1, g1, be1, W2, b2, g2, be2, W3, b3, g3, be3,
           W4, b4, g4, be4, Wo, bo):
    n = x.shape[0]
    e = edge_index.shape[1]
    nrows = ((n + NS * 8) // (NS * 8)) * (NS * 8)
    ept = GRP * CH
    ngrp = -(-e // (NW * ept))
    e_pad = NW * ngrp * ept

    src = jnp.concatenate([edge_index[0], jnp.zeros((e_pad - e,), jnp.int32)])
    dst = jnp.concatenate([edge_index[1], jnp.full((e_pad - e,), n, jnp.int32)])
    src3 = src.reshape(NW, ngrp * GRP, CH)
    dst3 = dst.reshape(NW, ngrp * GRP, CH)

    agg = _sc_agg_call(nrows, ngrp)

    ones_tab = jnp.ones((n, D), F32)
    degp = agg(ones_tab, src3, dst3)

    m1, dinv = pl.pallas_call(
        _pre_body,
        out_shape=[jax.ShapeDtypeStruct((n, D), F32),
                   jax.ShapeDtypeStruct((n, 1), F32)],
    )(x, _padw(W1, rows=x.shape[1]), degp)

    mid = pl.pallas_call(
        _mid_body,
        out_shape=jax.ShapeDtypeStruct((n, D), F32),
    )
    layers = [
        (m1, _padv(b1), _padv(g1), _padv(be1), _padw(W2)),
        (None, _padv(b2), _padv(g2), _padv(be2), _padw(W3)),
        (None, _padv(b3), _padv(g3), _padv(be3), _padw(W4)),
    ]
    m = m1
    for _, bv, gv, bev, wv in layers:
        aggp = agg(m, src3, dst3)
        m = mid(aggp, m, dinv, bv, gv, bev, wv)

    aggp = agg(m, src3, dst3)
    out2d = pl.pallas_call(
        _fin_body,
        out_shape=jax.ShapeDtypeStruct((1, 1), F32),
    )(aggp, m, dinv, _padv(b4), _padv(g4), _padv(be4),
      _padv(Wo[:, 0]), bo.reshape(1, 1))
    return out2d[:, 0]


# speedup vs baseline: 31.6486x; 1.1996x over previous
import functools

import jax
import jax.numpy as jnp
from jax import lax
from jax.experimental import pallas as pl
from jax.experimental.pallas import tpu as pltpu
from jax.experimental.pallas import tpu_sc as plsc

F32 = jnp.float32
EPS = 1e-5

NC = 2
NS = 16
NW = NC * NS
CH = 128
GRP = 8
D = 16


def _sc_agg_call(nrows, ngrp):
    zr = nrows // NS
    mesh = plsc.VectorSubcoreMesh(core_axis_name="c", subcore_axis_name="s")

    @functools.partial(
        pl.kernel,
        out_type=jax.ShapeDtypeStruct((NC, nrows, D), F32),
        mesh=mesh,
        scratch_types=[
            pltpu.VMEM_SHARED((nrows, D), F32),
            pltpu.VMEM((GRP, CH), jnp.int32),
            pltpu.VMEM((GRP, CH), jnp.int32),
            pltpu.VMEM((GRP, CH), jnp.int32),
            pltpu.VMEM((GRP, CH), jnp.int32),
            pltpu.VMEM((GRP, CH, D), F32),
            pltpu.VMEM((GRP, CH, D), F32),
            pltpu.VMEM((zr, D), F32),
            pltpu.SemaphoreType.DMA,
            pltpu.SemaphoreType.DMA,
            pltpu.SemaphoreType.DMA,
        ],
        compiler_params=pltpu.CompilerParams(use_tc_tiling_on_sc=False),
    )
    def agg(table, src3, dst3, out, acc, si0, di0, si1, di1, r0, r1, zb,
            gsem, ssem, isem):
        c = lax.axis_index("c")
        s = lax.axis_index("s")
        w = c * NS + s

        def zfill(i, _):
            zb[i] = jnp.zeros((D,), F32)
            return 0

        lax.fori_loop(0, zr, zfill, 0)
        row0 = pl.multiple_of(s * zr, 8)
        pltpu.sync_copy(zb, acc.at[pl.ds(row0, zr)])
        plsc.subcore_barrier()

        si = [si0, si1]
        di = [di0, di1]
        rows = [r0, r1]

        def load_idx(g, b):
            return (pltpu.async_copy(src3.at[w, pl.ds(g * GRP, GRP)], si[b], isem),
                    pltpu.async_copy(dst3.at[w, pl.ds(g * GRP, GRP)], di[b], isem))

        def fire_gathers(b):
            return [pltpu.async_copy(table.at[si[b].at[j]], rows[b].at[j], gsem)
                    for j in range(GRP)]

        def fire_scatters(b, gd):
            out_d = []
            for j in range(GRP):
                gd[j].wait()
                out_d.append(pltpu.async_copy(rows[b].at[j], acc.at[di[b].at[j]],
                                              ssem, add=True))
            return out_d

        pltpu.sync_copy(src3.at[w, pl.ds(0, GRP)], si0)
        pltpu.sync_copy(dst3.at[w, pl.ds(0, GRP)], di0)

        def body(k, _):
            g0 = k * 2
            gd0 = fire_gathers(0)
            ia, ib_ = load_idx(g0 + 1, 1)
            sd0 = fire_scatters(0, gd0)
            ia.wait(); ib_.wait()
            gd1 = fire_gathers(1)
            for d_ in sd0:
                d_.wait()
            ic, id_ = load_idx(g0 + 2, 0)
            sd1 = fire_scatters(1, gd1)
            ic.wait(); id_.wait()
            for d_ in sd1:
                d_.wait()
            return 0

        lax.fori_loop(0, ngrp // 2, body, 0)
        plsc.subcore_barrier()
        pltpu.sync_copy(acc.at[pl.ds(row0, zr)], out.at[c, pl.ds(row0, zr)])

    return agg


def _sc_deg_call(nrows, ngrp):
    zr = nrows // NS
    mesh = plsc.VectorSubcoreMesh(core_axis_name="c", subcore_axis_name="s")

    @functools.partial(
        pl.kernel,
        out_type=jax.ShapeDtypeStruct((NC, nrows), F32),
        mesh=mesh,
        scratch_types=[
            pltpu.VMEM_SHARED((nrows,), F32),
            pltpu.VMEM((GRP, CH), jnp.int32),
            pltpu.VMEM((GRP, CH), jnp.int32),
            pltpu.VMEM((CH,), F32),
            pltpu.VMEM((zr,), F32),
            pltpu.SemaphoreType.DMA,
            pltpu.SemaphoreType.DMA,
        ],
        compiler_params=pltpu.CompilerParams(use_tc_tiling_on_sc=False),
    )
    def deg(dst3, out, acc, di0, di1, ones, zb, ssem, isem):
        c = lax.axis_index("c")
        s = lax.axis_index("s")
        w = c * NS + s

        def zfill(i, _):
            zb[pl.ds(i * 16, 16)] = jnp.zeros((16,), F32)
            return 0

        lax.fori_loop(0, zr // 16, zfill, 0)
        for i in range(CH // 16):
            ones[pl.ds(i * 16, 16)] = jnp.ones((16,), F32)
        row0 = pl.multiple_of(s * zr, 8)
        pltpu.sync_copy(zb, acc.at[pl.ds(row0, zr)])
        plsc.subcore_barrier()

        di = [di0, di1]

        def fire_scatters(b):
            return [pltpu.async_copy(ones, acc.at[di[b].at[j]], ssem, add=True)
                    for j in range(GRP)]

        pltpu.sync_copy(dst3.at[w, pl.ds(0, GRP)], di0)

        def body(k, _):
            g0 = k * 2
            sd0 = fire_scatters(0)
            ia = pltpu.async_copy(dst3.at[w, pl.ds((g0 + 1) * GRP, GRP)], di1, isem)
            ia.wait()
            for d_ in sd0:
                d_.wait()
            sd1 = fire_scatters(1)
            ib_ = pltpu.async_copy(dst3.at[w, pl.ds((g0 + 2) * GRP, GRP)], di0, isem)
            ib_.wait()
            for d_ in sd1:
                d_.wait()
            return 0

        lax.fori_loop(0, ngrp // 2, body, 0)
        plsc.subcore_barrier()
        pltpu.sync_copy(acc.at[pl.ds(row0, zr)], out.at[c, pl.ds(row0, zr)])

    return deg


def _pre_body(x_ref, w1_ref, degp_ref, m_ref, dinv_ref):
    n = m_ref.shape[0]
    deg = degp_ref[0, :n, 0:1] + degp_ref[1, :n, 0:1] + 1.0
    dinv = lax.rsqrt(deg)
    h = jnp.dot(x_ref[...], w1_ref[...], preferred_element_type=F32)
    dinv_ref[...] = dinv
    m_ref[...] = h * dinv


def _bn_relu(aggp_ref, m_ref, dinv_ref, b_ref, g_ref, be_ref):
    n = m_ref.shape[0]
    dinv = dinv_ref[...]
    t = (aggp_ref[0, :n] + aggp_ref[1, :n] + m_ref[...]) * dinv + b_ref[...]
    mu = jnp.mean(t, axis=0, keepdims=True)
    var = jnp.mean((t - mu) ** 2, axis=0, keepdims=True)
    return jnp.maximum((t - mu) * lax.rsqrt(var + EPS) * g_ref[...] + be_ref[...], 0.0)


def _mid_body(aggp_ref, m_ref, dinv_ref, b_ref, g_ref, be_ref, w_ref, out_ref):
    y = _bn_relu(aggp_ref, m_ref, dinv_ref, b_ref, g_ref, be_ref)
    out_ref[...] = jnp.dot(y, w_ref[...], preferred_element_type=F32) * dinv_ref[...]


def _fin_body(aggp_ref, m_ref, dinv_ref, b_ref, g_ref, be_ref, wo_ref, bo_ref, out_ref):
    y = _bn_relu(aggp_ref, m_ref, dinv_ref, b_ref, g_ref, be_ref)
    pooled = jnp.max(y, axis=0, keepdims=True)
    out_ref[...] = jnp.sum(pooled * wo_ref[...], axis=1, keepdims=True) + bo_ref[...]


def _padw(w, rows=D, cols=D):
    return jnp.pad(w, ((0, rows - w.shape[0]), (0, cols - w.shape[1])))


def _padv(v):
    return jnp.pad(v, (0, D - v.shape[0])).reshape(1, D)


def kernel(x, edge_index, W1, b1, g1, be1, W2, b2, g2, be2, W3, b3, g3, be3,
           W4, b4, g4, be4, Wo, bo):
    n = x.shape[0]
    e = edge_index.shape[1]
    nrows = ((n + 1 + 255) // 256) * 256
    ept = GRP * CH
    ngrp = -(-e // (NW * ept))
    ngrp += ngrp % 2
    e_pad = NW * ngrp * ept

    src = jnp.concatenate([edge_index[0], jnp.zeros((e_pad - e,), jnp.int32)])
    dst = jnp.concatenate([edge_index[1], jnp.full((e_pad - e,), n, jnp.int32)])
    dummy = jnp.zeros((NW, GRP, CH), jnp.int32)
    src3 = jnp.concatenate([src.reshape(NW, ngrp * GRP, CH), dummy], axis=1)
    dst3 = jnp.concatenate([dst.reshape(NW, ngrp * GRP, CH), dummy], axis=1)

    agg = _sc_agg_call(nrows, ngrp)
    degp = _sc_deg_call(nrows, ngrp)(dst3)

    m1, dinv = pl.pallas_call(
        _pre_body,
        out_shape=[jax.ShapeDtypeStruct((n, D), F32),
                   jax.ShapeDtypeStruct((n, 1), F32)],
    )(x, _padw(W1, rows=x.shape[1]), degp.reshape(NC, nrows, 1))

    mid = pl.pallas_call(
        _mid_body,
        out_shape=jax.ShapeDtypeStruct((n, D), F32),
    )
    layers = [
        (_padv(b1), _padv(g1), _padv(be1), _padw(W2)),
        (_padv(b2), _padv(g2), _padv(be2), _padw(W3)),
        (_padv(b3), _padv(g3), _padv(be3), _padw(W4)),
    ]
    m = m1
    for bv, gv, bev, wv in layers:
        aggp = agg(m, src3, dst3)
        m = mid(aggp, m, dinv, bv, gv, bev, wv)

    aggp = agg(m, src3, dst3)
    out2d = pl.pallas_call(
        _fin_body,
        out_shape=jax.ShapeDtypeStruct((1, 1), F32),
    )(aggp, m, dinv, _padv(b4), _padv(g4), _padv(be4),
      _padv(Wo[:, 0]), bo.reshape(1, 1))
    return out2d[:, 0]
